# baseline (device time: 48157 ns/iter reference)
import functools

import jax
import jax.numpy as jnp
from jax import lax
from jax.experimental import pallas as pl
from jax.experimental.pallas import tpu as pltpu

N_DEV = 32
CHUNK = 16

STRIPES = (
    (0, 768, (1, 3, 0, 2, 4)),
    (768, 640, (3, 0, 1, 4, 2)),
    (1408, 640, (0, 1, 3, 4, 2)),
)
N_PHASES = 5
RECV_ROWS = tuple(256 >> j for j in range(N_PHASES))


def _rank_of_l(l):
    z = l // 8
    rem = l % 8
    y = rem // 2
    xx = rem % 2
    q = jnp.bitwise_xor(xx, y % 2)
    return 8 * z + 2 * y + q


def _l_of_rank(r):
    z = r // 8
    p = r % 8
    y = p // 2
    q = p % 2
    xx = jnp.bitwise_xor(q, y % 2)
    return 8 * z + 2 * y + xx


def _stripe_masks(order):
    masks = []
    for idx in range(32):
        mask = 0
        for j in range(N_PHASES):
            if (idx >> (4 - j)) & 1:
                mask |= 1 << order[j]
        masks.append(mask)
    return masks


def kernel(x, dy):
    m, d_in = x.shape
    _, d_out = dy.shape

    my = lax.axis_index("i")
    my_l = _l_of_rank(my)

    xps = []
    for _, _, order in STRIPES:
        masks = jnp.array(_stripe_masks(order), dtype=jnp.int32)
        c_rank = _rank_of_l(jnp.bitwise_xor(my_l, masks))
        cols = (c_rank[:, None] * CHUNK
                + jnp.arange(CHUNK, dtype=jnp.int32)[None, :]).reshape(-1)
        xps.append(jnp.take(x, cols, axis=1))

    def body(xp0_ref, xp1_ref, xp2_ref, dy_ref, out_ref, *scratch):
        xp_refs = (xp0_ref, xp1_ref, xp2_ref)
        wbufs = scratch[0:3]
        rbufs = scratch[3:3 + 3 * N_PHASES]
        send_sems, recv_sems = scratch[3 + 3 * N_PHASES:]

        my_l_in = _l_of_rank(lax.axis_index("i"))
        partner = [
            _rank_of_l(jnp.bitwise_xor(my_l_in, 1 << b)) for b in range(5)
        ]

        barrier_sem = pltpu.get_barrier_semaphore()
        for pr in partner:
            pl.semaphore_signal(
                barrier_sem, inc=1,
                device_id=(pr,), device_id_type=pl.DeviceIdType.MESH,
            )
        pl.semaphore_wait(barrier_sem, 5)

        def exchange(s, j, start_only):
            _, _, order = STRIPES[s]
            rows = RECV_ROWS[j]
            rdma = pltpu.make_async_remote_copy(
                src_ref=wbufs[s].at[pl.ds(rows, rows), :],
                dst_ref=rbufs[N_PHASES * s + j],
                send_sem=send_sems.at[s, j],
                recv_sem=recv_sems.at[s, j],
                device_id=(partner[order[j]],),
                device_id_type=pl.DeviceIdType.MESH,
            )
            if start_only:
                rdma.start()
            return rdma

        sends = []
        for s in range(3):
            c0, w, _ = STRIPES[s]
            wbufs[s][...] = lax.dot_general(
                xp_refs[s][...], dy_ref[:, pl.ds(c0, w)],
                dimension_numbers=(((0,), (0,)), ((), ())),
                preferred_element_type=jnp.float32,
            )
            sends.append(exchange(s, 0, True))

        for j in range(N_PHASES):
            rows = RECV_ROWS[j]
            for s in range(3):
                exchange(s, j, False).wait_recv()
                wbufs[s][pl.ds(0, rows), :] = (
                    wbufs[s][pl.ds(0, rows), :] + rbufs[N_PHASES * s + j][...]
                )
                if j + 1 < N_PHASES:
                    sends.append(exchange(s, j + 1, True))

        for s in range(3):
            c0, w, _ = STRIPES[s]
            out_ref[:, pl.ds(c0, w)] = wbufs[s][pl.ds(0, CHUNK), :]

        for rdma in sends:
            rdma.wait_send()

        @functools.partial(pl.run_scoped, exit_sem=pltpu.SemaphoreType.REGULAR)
        def _(exit_sem):
            for pr in partner:
                pl.semaphore_signal(
                    exit_sem, inc=1,
                    device_id=(pr,), device_id_type=pl.DeviceIdType.MESH,
                )
            pl.semaphore_wait(exit_sem, 5)

    scratch_shapes = [
        pltpu.VMEM((m, w), jnp.float32) for (_, w, _) in STRIPES
    ]
    for (_, w, _) in STRIPES:
        scratch_shapes += [
            pltpu.VMEM((RECV_ROWS[j], w), jnp.float32) for j in range(N_PHASES)
        ]
    scratch_shapes += [
        pltpu.SemaphoreType.DMA((3, N_PHASES)),
        pltpu.SemaphoreType.DMA((3, N_PHASES)),
    ]

    return pl.pallas_call(
        body,
        out_shape=jax.ShapeDtypeStruct((CHUNK, d_out), jnp.float32),
        in_specs=[pl.BlockSpec(memory_space=pltpu.VMEM)] * 4,
        out_specs=pl.BlockSpec(memory_space=pltpu.VMEM),
        scratch_shapes=scratch_shapes,
        compiler_params=pltpu.CompilerParams(collective_id=0),
    )(*xps, dy)


# device time: 43570 ns/iter; 1.1053x vs baseline; 1.1053x over previous
import functools

import jax
import jax.numpy as jnp
from jax import lax
from jax.experimental import pallas as pl
from jax.experimental.pallas import tpu as pltpu

N_DEV = 32
CHUNK = 16

STRIPES = (
    (0, 768, (1, 3, 0, 2, 4)),
    (768, 640, (3, 0, 1, 4, 2)),
    (1408, 640, (0, 1, 3, 4, 2)),
)
N_PHASES = 5
RECV_ROWS = tuple(256 >> j for j in range(N_PHASES))


def _rank_of_l(l):
    z = l // 8
    rem = l % 8
    y = rem // 2
    xx = rem % 2
    q = jnp.bitwise_xor(xx, y % 2)
    return 8 * z + 2 * y + q


def _l_of_rank(r):
    z = r // 8
    p = r % 8
    y = p // 2
    q = p % 2
    xx = jnp.bitwise_xor(q, y % 2)
    return 8 * z + 2 * y + xx


def _stripe_masks(order):
    masks = []
    for idx in range(32):
        mask = 0
        for j in range(N_PHASES):
            if (idx >> (4 - j)) & 1:
                mask |= 1 << order[j]
        masks.append(mask)
    return masks


def kernel(x, dy):
    m, d_in = x.shape
    _, d_out = dy.shape

    my = lax.axis_index("i")
    my_l = _l_of_rank(my)

    xps = []
    for _, _, order in STRIPES:
        masks = jnp.array(_stripe_masks(order), dtype=jnp.int32)
        c_rank = _rank_of_l(jnp.bitwise_xor(my_l, masks))
        xps.append(
            jnp.take(x.reshape(m, N_DEV, CHUNK), c_rank, axis=1)
            .reshape(m, d_in)
        )

    def body(xp0_ref, xp1_ref, xp2_ref, dy_ref, out_ref, *scratch):
        xp_refs = (xp0_ref, xp1_ref, xp2_ref)
        wbufs = scratch[0:3]
        rbufs = scratch[3:3 + 3 * N_PHASES]
        send_sems, recv_sems = scratch[3 + 3 * N_PHASES:]

        my_l_in = _l_of_rank(lax.axis_index("i"))
        partner = [
            _rank_of_l(jnp.bitwise_xor(my_l_in, 1 << b)) for b in range(5)
        ]

        barrier_sem = pltpu.get_barrier_semaphore()
        for pr in partner:
            pl.semaphore_signal(
                barrier_sem, inc=1,
                device_id=(pr,), device_id_type=pl.DeviceIdType.MESH,
            )
        pl.semaphore_wait(barrier_sem, 5)

        def exchange(s, j, start_only):
            _, _, order = STRIPES[s]
            rows = RECV_ROWS[j]
            rdma = pltpu.make_async_remote_copy(
                src_ref=wbufs[s].at[pl.ds(rows, rows), :],
                dst_ref=rbufs[N_PHASES * s + j],
                send_sem=send_sems.at[s, j],
                recv_sem=recv_sems.at[s, j],
                device_id=(partner[order[j]],),
                device_id_type=pl.DeviceIdType.MESH,
            )
            if start_only:
                rdma.start()
            return rdma

        sends = []
        for s in range(3):
            c0, w, _ = STRIPES[s]
            wbufs[s][...] = lax.dot_general(
                xp_refs[s][...], dy_ref[:, pl.ds(c0, w)],
                dimension_numbers=(((0,), (0,)), ((), ())),
                preferred_element_type=jnp.float32,
            )
            sends.append(exchange(s, 0, True))

        for j in range(N_PHASES):
            rows = RECV_ROWS[j]
            for s in range(3):
                exchange(s, j, False).wait_recv()
                wbufs[s][pl.ds(0, rows), :] = (
                    wbufs[s][pl.ds(0, rows), :] + rbufs[N_PHASES * s + j][...]
                )
                if j + 1 < N_PHASES:
                    sends.append(exchange(s, j + 1, True))

        for s in range(3):
            c0, w, _ = STRIPES[s]
            out_ref[:, pl.ds(c0, w)] = wbufs[s][pl.ds(0, CHUNK), :]

        for rdma in sends:
            rdma.wait_send()

        @functools.partial(pl.run_scoped, exit_sem=pltpu.SemaphoreType.REGULAR)
        def _(exit_sem):
            for pr in partner:
                pl.semaphore_signal(
                    exit_sem, inc=1,
                    device_id=(pr,), device_id_type=pl.DeviceIdType.MESH,
                )
            pl.semaphore_wait(exit_sem, 5)

    scratch_shapes = [
        pltpu.VMEM((m, w), jnp.float32) for (_, w, _) in STRIPES
    ]
    for (_, w, _) in STRIPES:
        scratch_shapes += [
            pltpu.VMEM((RECV_ROWS[j], w), jnp.float32) for j in range(N_PHASES)
        ]
    scratch_shapes += [
        pltpu.SemaphoreType.DMA((3, N_PHASES)),
        pltpu.SemaphoreType.DMA((3, N_PHASES)),
    ]

    return pl.pallas_call(
        body,
        out_shape=jax.ShapeDtypeStruct((CHUNK, d_out), jnp.float32),
        in_specs=[pl.BlockSpec(memory_space=pltpu.VMEM)] * 4,
        out_specs=pl.BlockSpec(memory_space=pltpu.VMEM),
        scratch_shapes=scratch_shapes,
        compiler_params=pltpu.CompilerParams(collective_id=0),
    )(*xps, dy)


# device time: 41313 ns/iter; 1.1657x vs baseline; 1.0546x over previous
import functools

import jax
import jax.numpy as jnp
import numpy as np
from jax import lax
from jax.experimental import pallas as pl
from jax.experimental.pallas import tpu as pltpu

N_DEV = 32
CHUNK = 16

STRIPES = (
    (0, 768, (1, 3, 0, 2, 4)),
    (768, 640, (3, 0, 1, 4, 2)),
    (1408, 640, (0, 1, 3, 4, 2)),
)
N_PHASES = 5
RECV_ROWS = tuple(256 >> j for j in range(N_PHASES))


def _rank_of_l(l):
    z = l // 8
    rem = l % 8
    y = rem // 2
    xx = rem % 2
    q = jnp.bitwise_xor(xx, y % 2)
    return 8 * z + 2 * y + q


def _l_of_rank(r):
    z = r // 8
    p = r % 8
    y = p // 2
    q = p % 2
    xx = jnp.bitwise_xor(q, y % 2)
    return 8 * z + 2 * y + xx


def _stripe_col_masks(order):
    masks = []
    for idx in range(N_DEV):
        mask = 0
        for j in range(N_PHASES):
            if (idx >> (4 - j)) & 1:
                mask |= 1 << order[j]
        masks.append(mask)
    return np.repeat(np.array(masks, np.int32), CHUNK)


def kernel(x, dy):
    m, d_in = x.shape
    _, d_out = dy.shape

    def body(x_ref, dy_ref, out_ref, *scratch):
        wbufs = scratch[0:3]
        xp_ref = scratch[3]
        rbufs = scratch[4:4 + 3 * N_PHASES]
        send_sems, recv_sems = scratch[4 + 3 * N_PHASES:]

        my = lax.axis_index("i")
        my_l = _l_of_rank(my)
        partner = [_rank_of_l(jnp.bitwise_xor(my_l, 1 << b)) for b in range(5)]

        barrier_sem = pltpu.get_barrier_semaphore()
        for pr in partner:
            pl.semaphore_signal(
                barrier_sem, inc=1,
                device_id=(pr,), device_id_type=pl.DeviceIdType.MESH,
            )
        pl.semaphore_wait(barrier_sem, 5)

        def exchange(s, j, start_only):
            _, _, order = STRIPES[s]
            rows = RECV_ROWS[j]
            rdma = pltpu.make_async_remote_copy(
                src_ref=wbufs[s].at[pl.ds(rows, rows), :],
                dst_ref=rbufs[N_PHASES * s + j],
                send_sem=send_sems.at[s, j],
                recv_sem=recv_sems.at[s, j],
                device_id=(partner[order[j]],),
                device_id_type=pl.DeviceIdType.MESH,
            )
            if start_only:
                rdma.start()
            return rdma

        row_iota = lax.broadcasted_iota(jnp.int32, (d_in, d_in), 0)
        col_iota = lax.broadcasted_iota(jnp.int32, (1, d_in), 1)
        pos = col_iota // CHUNK
        within = col_iota % CHUNK

        sends = []
        for s in range(3):
            c0, w, order = STRIPES[s]
            mask = jnp.zeros_like(pos)
            for j in range(N_PHASES):
                kj = (pos // (1 << (4 - j))) % 2
                mask = mask + kj * (1 << order[j])
            c_l = jnp.bitwise_xor(my_l, mask)
            cols = _rank_of_l(c_l) * CHUNK + within
            m_s = (row_iota == cols).astype(jnp.float32)
            xp_ref[...] = lax.dot_general(
                x_ref[...], m_s,
                dimension_numbers=(((1,), (0,)), ((), ())),
                preferred_element_type=jnp.float32,
            )
            wbufs[s][...] = lax.dot_general(
                xp_ref[...], dy_ref[:, pl.ds(c0, w)],
                dimension_numbers=(((0,), (0,)), ((), ())),
                preferred_element_type=jnp.float32,
            )
            sends.append(exchange(s, 0, True))

        for j in range(N_PHASES):
            rows = RECV_ROWS[j]
            for s in range(3):
                exchange(s, j, False).wait_recv()
                wbufs[s][pl.ds(0, rows), :] = (
                    wbufs[s][pl.ds(0, rows), :] + rbufs[N_PHASES * s + j][...]
                )
                if j + 1 < N_PHASES:
                    sends.append(exchange(s, j + 1, True))

        for s in range(3):
            c0, w, _ = STRIPES[s]
            out_ref[:, pl.ds(c0, w)] = wbufs[s][pl.ds(0, CHUNK), :]

        for rdma in sends:
            rdma.wait_send()

        @functools.partial(pl.run_scoped, exit_sem=pltpu.SemaphoreType.REGULAR)
        def _(exit_sem):
            for pr in partner:
                pl.semaphore_signal(
                    exit_sem, inc=1,
                    device_id=(pr,), device_id_type=pl.DeviceIdType.MESH,
                )
            pl.semaphore_wait(exit_sem, 5)

    scratch_shapes = [
        pltpu.VMEM((m, w), jnp.float32) for (_, w, _) in STRIPES
    ]
    scratch_shapes += [pltpu.VMEM((m, d_in), jnp.float32)]
    for (_, w, _) in STRIPES:
        scratch_shapes += [
            pltpu.VMEM((RECV_ROWS[j], w), jnp.float32) for j in range(N_PHASES)
        ]
    scratch_shapes += [
        pltpu.SemaphoreType.DMA((3, N_PHASES)),
        pltpu.SemaphoreType.DMA((3, N_PHASES)),
    ]

    return pl.pallas_call(
        body,
        out_shape=jax.ShapeDtypeStruct((CHUNK, d_out), jnp.float32),
        in_specs=[
            pl.BlockSpec(memory_space=pltpu.VMEM),
            pl.BlockSpec(memory_space=pltpu.VMEM),
        ],
        out_specs=pl.BlockSpec(memory_space=pltpu.VMEM),
        scratch_shapes=scratch_shapes,
        compiler_params=pltpu.CompilerParams(collective_id=0),
    )(x, dy)


# device time: 39873 ns/iter; 1.2078x vs baseline; 1.0361x over previous
import functools

import jax
import jax.numpy as jnp
import numpy as np
from jax import lax
from jax.experimental import pallas as pl
from jax.experimental.pallas import tpu as pltpu

N_DEV = 32
CHUNK = 16

STRIPES = (
    (0, 768, (1, 3, 0, 2, 4)),
    (768, 640, (3, 0, 1, 4, 2)),
    (1408, 640, (0, 1, 3, 4, 2)),
)
N_PHASES = 5
RECV_ROWS = tuple(256 >> j for j in range(N_PHASES))


def _rank_of_l(l):
    z = l // 8
    rem = l % 8
    y = rem // 2
    xx = rem % 2
    q = jnp.bitwise_xor(xx, y % 2)
    return 8 * z + 2 * y + q


def _l_of_rank(r):
    z = r // 8
    p = r % 8
    y = p // 2
    q = p % 2
    xx = jnp.bitwise_xor(q, y % 2)
    return 8 * z + 2 * y + xx


def _stripe_col_masks(order):
    masks = []
    for idx in range(N_DEV):
        mask = 0
        for j in range(N_PHASES):
            if (idx >> (4 - j)) & 1:
                mask |= 1 << order[j]
        masks.append(mask)
    return np.repeat(np.array(masks, np.int32), CHUNK)


def kernel(x, dy):
    m, d_in = x.shape
    _, d_out = dy.shape

    def body(x_ref, dy_ref, out_ref, *scratch):
        wbufs = scratch[0:3]
        xp_ref = scratch[3]
        rbufs = scratch[4:4 + 3 * N_PHASES]
        send_sems, recv_sems = scratch[4 + 3 * N_PHASES:]

        my = lax.axis_index("i")
        my_l = _l_of_rank(my)
        partner = [_rank_of_l(jnp.bitwise_xor(my_l, 1 << b)) for b in range(5)]

        barrier_sem = pltpu.get_barrier_semaphore()
        for pr in partner:
            pl.semaphore_signal(
                barrier_sem, inc=1,
                device_id=(pr,), device_id_type=pl.DeviceIdType.MESH,
            )

        def exchange(s, j, start_only):
            _, _, order = STRIPES[s]
            rows = RECV_ROWS[j]
            rdma = pltpu.make_async_remote_copy(
                src_ref=wbufs[s].at[pl.ds(rows, rows), :],
                dst_ref=rbufs[N_PHASES * s + j],
                send_sem=send_sems.at[s, j],
                recv_sem=recv_sems.at[s, j],
                device_id=(partner[order[j]],),
                device_id_type=pl.DeviceIdType.MESH,
            )
            if start_only:
                rdma.start()
            return rdma

        row_iota = lax.broadcasted_iota(jnp.int32, (d_in, d_in), 0)
        col_iota = lax.broadcasted_iota(jnp.int32, (1, d_in), 1)
        pos = col_iota // CHUNK
        within = col_iota % CHUNK

        sends = []
        for s in range(3):
            c0, w, order = STRIPES[s]
            mask = jnp.zeros_like(pos)
            for j in range(N_PHASES):
                kj = (pos // (1 << (4 - j))) % 2
                mask = mask + kj * (1 << order[j])
            c_l = jnp.bitwise_xor(my_l, mask)
            cols = _rank_of_l(c_l) * CHUNK + within
            m_s = (row_iota == cols).astype(jnp.float32)
            xp_ref[...] = lax.dot_general(
                x_ref[...], m_s,
                dimension_numbers=(((1,), (0,)), ((), ())),
                preferred_element_type=jnp.float32,
            )
            wbufs[s][...] = lax.dot_general(
                xp_ref[...], dy_ref[:, pl.ds(c0, w)],
                dimension_numbers=(((0,), (0,)), ((), ())),
                preferred_element_type=jnp.float32,
            )
            if s == 0:
                pl.semaphore_wait(barrier_sem, 5)
            sends.append(exchange(s, 0, True))

        for j in range(N_PHASES):
            rows = RECV_ROWS[j]
            for s in range(3):
                exchange(s, j, False).wait_recv()
                wbufs[s][pl.ds(0, rows), :] = (
                    wbufs[s][pl.ds(0, rows), :] + rbufs[N_PHASES * s + j][...]
                )
                if j + 1 < N_PHASES:
                    sends.append(exchange(s, j + 1, True))

        for s in range(3):
            c0, w, _ = STRIPES[s]
            out_ref[:, pl.ds(c0, w)] = wbufs[s][pl.ds(0, CHUNK), :]

        for rdma in sends:
            rdma.wait_send()

        @functools.partial(pl.run_scoped, exit_sem=pltpu.SemaphoreType.REGULAR)
        def _(exit_sem):
            for pr in partner:
                pl.semaphore_signal(
                    exit_sem, inc=1,
                    device_id=(pr,), device_id_type=pl.DeviceIdType.MESH,
                )
            pl.semaphore_wait(exit_sem, 5)

    scratch_shapes = [
        pltpu.VMEM((m, w), jnp.float32) for (_, w, _) in STRIPES
    ]
    scratch_shapes += [pltpu.VMEM((m, d_in), jnp.float32)]
    for (_, w, _) in STRIPES:
        scratch_shapes += [
            pltpu.VMEM((RECV_ROWS[j], w), jnp.float32) for j in range(N_PHASES)
        ]
    scratch_shapes += [
        pltpu.SemaphoreType.DMA((3, N_PHASES)),
        pltpu.SemaphoreType.DMA((3, N_PHASES)),
    ]

    return pl.pallas_call(
        body,
        out_shape=jax.ShapeDtypeStruct((CHUNK, d_out), jnp.float32),
        in_specs=[
            pl.BlockSpec(memory_space=pltpu.VMEM),
            pl.BlockSpec(memory_space=pltpu.VMEM),
        ],
        out_specs=pl.BlockSpec(memory_space=pltpu.VMEM),
        scratch_shapes=scratch_shapes,
        compiler_params=pltpu.CompilerParams(collective_id=0),
    )(x, dy)


# device time: 39800 ns/iter; 1.2100x vs baseline; 1.0018x over previous
import functools

import jax
import jax.numpy as jnp
import numpy as np
from jax import lax
from jax.experimental import pallas as pl
from jax.experimental.pallas import tpu as pltpu

N_DEV = 32
CHUNK = 16

STRIPES = (
    (0, 768, (1, 3, 0, 2, 4)),
    (768, 640, (3, 0, 1, 4, 2)),
    (1408, 640, (0, 1, 3, 4, 2)),
)
N_PHASES = 5
RECV_ROWS = tuple(256 >> j for j in range(N_PHASES))


def _rank_of_l(l):
    z = l // 8
    rem = l % 8
    y = rem // 2
    xx = rem % 2
    q = jnp.bitwise_xor(xx, y % 2)
    return 8 * z + 2 * y + q


def _l_of_rank(r):
    z = r // 8
    p = r % 8
    y = p // 2
    q = p % 2
    xx = jnp.bitwise_xor(q, y % 2)
    return 8 * z + 2 * y + xx


def _stripe_col_masks(order):
    masks = []
    for idx in range(N_DEV):
        mask = 0
        for j in range(N_PHASES):
            if (idx >> (4 - j)) & 1:
                mask |= 1 << order[j]
        masks.append(mask)
    return np.repeat(np.array(masks, np.int32), CHUNK)


def kernel(x, dy):
    m, d_in = x.shape
    _, d_out = dy.shape

    def body(x_ref, dy_ref, out_ref, *scratch):
        wbufs = scratch[0:3]
        xp_ref = scratch[3]
        rbufs = scratch[4:4 + 3 * N_PHASES]
        send_sems, recv_sems = scratch[4 + 3 * N_PHASES:]

        my = lax.axis_index("i")
        my_l = _l_of_rank(my)
        partner = [_rank_of_l(jnp.bitwise_xor(my_l, 1 << b)) for b in range(5)]

        barrier_sem = pltpu.get_barrier_semaphore()
        for pr in partner:
            pl.semaphore_signal(
                barrier_sem, inc=1,
                device_id=(pr,), device_id_type=pl.DeviceIdType.MESH,
            )

        def exchange(s, j, start_only):
            _, _, order = STRIPES[s]
            rows = RECV_ROWS[j]
            rdma = pltpu.make_async_remote_copy(
                src_ref=wbufs[s].at[pl.ds(rows, rows), :],
                dst_ref=rbufs[N_PHASES * s + j],
                send_sem=send_sems.at[s, j],
                recv_sem=recv_sems.at[s, j],
                device_id=(partner[order[j]],),
                device_id_type=pl.DeviceIdType.MESH,
            )
            if start_only:
                rdma.start()
            return rdma

        row_iota = lax.broadcasted_iota(jnp.int32, (d_in, d_in), 0)
        col_iota = lax.broadcasted_iota(jnp.int32, (1, d_in), 1)
        pos = col_iota // CHUNK
        within = col_iota % CHUNK

        sends = []
        for s in range(3):
            c0, w, order = STRIPES[s]
            mask = jnp.zeros_like(pos)
            for j in range(N_PHASES):
                kj = (pos // (1 << (4 - j))) % 2
                mask = mask + kj * (1 << order[j])
            c_l = jnp.bitwise_xor(my_l, mask)
            cols = _rank_of_l(c_l) * CHUNK + within
            m_s = (row_iota == cols).astype(jnp.float32)
            xp_ref[...] = lax.dot_general(
                x_ref[...], m_s,
                dimension_numbers=(((1,), (0,)), ((), ())),
                preferred_element_type=jnp.float32,
            )
            wbufs[s][...] = lax.dot_general(
                xp_ref[...], dy_ref[:, pl.ds(c0, w)],
                dimension_numbers=(((0,), (0,)), ((), ())),
                preferred_element_type=jnp.float32,
            )
            if s == 0:
                pl.semaphore_wait(barrier_sem, 5)
            sends.append(exchange(s, 0, True))

        for j in range(N_PHASES):
            rows = RECV_ROWS[j]
            half = rows // 2
            for s in range(3):
                exchange(s, j, False).wait_recv()
                rb = rbufs[N_PHASES * s + j]
                if j + 1 < N_PHASES:
                    wbufs[s][pl.ds(half, half), :] = (
                        wbufs[s][pl.ds(half, half), :] + rb[pl.ds(half, half), :]
                    )
                    sends.append(exchange(s, j + 1, True))
                    wbufs[s][pl.ds(0, half), :] = (
                        wbufs[s][pl.ds(0, half), :] + rb[pl.ds(0, half), :]
                    )
                else:
                    wbufs[s][pl.ds(0, rows), :] = (
                        wbufs[s][pl.ds(0, rows), :] + rb[...]
                    )

        for s in range(3):
            c0, w, _ = STRIPES[s]
            out_ref[:, pl.ds(c0, w)] = wbufs[s][pl.ds(0, CHUNK), :]

        for rdma in sends:
            rdma.wait_send()

        @functools.partial(pl.run_scoped, exit_sem=pltpu.SemaphoreType.REGULAR)
        def _(exit_sem):
            for pr in partner:
                pl.semaphore_signal(
                    exit_sem, inc=1,
                    device_id=(pr,), device_id_type=pl.DeviceIdType.MESH,
                )
            pl.semaphore_wait(exit_sem, 5)

    scratch_shapes = [
        pltpu.VMEM((m, w), jnp.float32) for (_, w, _) in STRIPES
    ]
    scratch_shapes += [pltpu.VMEM((m, d_in), jnp.float32)]
    for (_, w, _) in STRIPES:
        scratch_shapes += [
            pltpu.VMEM((RECV_ROWS[j], w), jnp.float32) for j in range(N_PHASES)
        ]
    scratch_shapes += [
        pltpu.SemaphoreType.DMA((3, N_PHASES)),
        pltpu.SemaphoreType.DMA((3, N_PHASES)),
    ]

    return pl.pallas_call(
        body,
        out_shape=jax.ShapeDtypeStruct((CHUNK, d_out), jnp.float32),
        in_specs=[
            pl.BlockSpec(memory_space=pltpu.VMEM),
            pl.BlockSpec(memory_space=pltpu.VMEM),
        ],
        out_specs=pl.BlockSpec(memory_space=pltpu.VMEM),
        scratch_shapes=scratch_shapes,
        compiler_params=pltpu.CompilerParams(collective_id=0),
    )(x, dy)
